# Initial kernel scaffold; baseline (speedup 1.0000x reference)
import jax
import jax.numpy as jnp
from jax.experimental import pallas as pl


def kernel(bonds, bond_atom_1, bond_atom_2, atoms, state, e_W1, e_b1, e_W2, e_b2, e_W3, e_b3, v_W1, v_b1, v_W2, v_b2, v_W3, v_b3, u_W1, u_b1, u_W2, u_b2, u_W3, u_b3):
    raise NotImplementedError("write your pallas kernel here")


# speedup vs baseline: 3.2413x; 3.2413x over previous
import functools

import jax
import jax.numpy as jnp
from jax import lax
from jax.experimental import pallas as pl
from jax.experimental.pallas import tpu as pltpu
from jax.experimental.pallas import tpu_sc as plsc

NB = 1_600_000
NA = 100_000
D = 32
NC = 2
NS = 16
NW = NC * NS

GCHUNK = 2000
BONDS_PER_W = NB // NW
GCHUNKS = BONDS_PER_W // GCHUNK


def _gather_body(atoms_hbm, idx1_hbm, idx2_hbm, a1_hbm, a2_hbm,
                 idx_v, rows_v, sem):
    c = lax.axis_index("c")
    s = lax.axis_index("s")
    wid = s * NC + c
    base = wid * BONDS_PER_W

    def chunk(j, carry):
        off = base + j * GCHUNK
        pltpu.sync_copy(idx1_hbm.at[pl.ds(off, GCHUNK)], idx_v)
        pltpu.async_copy(atoms_hbm.at[idx_v], rows_v, sem).wait()
        pltpu.sync_copy(rows_v, a1_hbm.at[pl.ds(off, GCHUNK)])
        pltpu.sync_copy(idx2_hbm.at[pl.ds(off, GCHUNK)], idx_v)
        pltpu.async_copy(atoms_hbm.at[idx_v], rows_v, sem).wait()
        pltpu.sync_copy(rows_v, a2_hbm.at[pl.ds(off, GCHUNK)])
        return carry

    lax.fori_loop(0, GCHUNKS, chunk, 0)


def _sc_gather(atoms, idx1, idx2):
    mesh = plsc.VectorSubcoreMesh(core_axis_name="c", subcore_axis_name="s")
    f = pl.kernel(
        _gather_body,
        out_type=[jax.ShapeDtypeStruct((NB, D), jnp.float32),
                  jax.ShapeDtypeStruct((NB, D), jnp.float32)],
        mesh=mesh,
        compiler_params=pltpu.CompilerParams(use_tc_tiling_on_sc=False),
        scratch_types=[pltpu.VMEM((GCHUNK,), jnp.int32),
                       pltpu.VMEM((GCHUNK, D), jnp.float32),
                       pltpu.SemaphoreType.DMA],
    )
    return f(atoms, idx1, idx2)


ATOMS_PER_SC = NA // NC
ACC_ROWS = 50176
STRIPE = ACC_ROWS // NS
LAST_STRIPE = ATOMS_PER_SC - (NS - 1) * STRIPE
SCHUNK = 400
BONDS_PER_T = NB // NS
SCHUNKS = BONDS_PER_T // SCHUNK
VGRP = SCHUNK // 16


def _scatter_body(bnew_hbm, idx_hbm, seg_hbm, cnt_hbm,
                  idx_v, lidx_v, rows_v, ones_v, feat_acc, cnt_acc, sem):
    c = lax.axis_index("c")
    s = lax.axis_index("s")
    lo = c * ATOMS_PER_SC

    def zrow(i, carry):
        rows_v[i, pl.ds(0, 16)] = jnp.zeros((16,), jnp.float32)
        rows_v[i, pl.ds(16, 16)] = jnp.zeros((16,), jnp.float32)
        return carry

    lax.fori_loop(0, SCHUNK, zrow, 0)

    def zone(g, carry):
        ones_v[pl.ds(g * 16, 16)] = jnp.zeros((16,), jnp.float32)
        return carry

    lax.fori_loop(0, VGRP, zone, 0)

    nfull = STRIPE // SCHUNK
    rem = STRIPE - nfull * SCHUNK

    def zcopy(k, carry):
        pltpu.sync_copy(rows_v,
                        feat_acc.at[pl.ds(s * STRIPE + k * SCHUNK, SCHUNK)])
        pltpu.sync_copy(ones_v,
                        cnt_acc.at[pl.ds(s * STRIPE + k * SCHUNK, SCHUNK)])
        return carry

    lax.fori_loop(0, nfull, zcopy, 0)
    pltpu.sync_copy(rows_v.at[pl.ds(0, rem)],
                    feat_acc.at[pl.ds(s * STRIPE + nfull * SCHUNK, rem)])
    pltpu.sync_copy(ones_v.at[pl.ds(0, rem)],
                    cnt_acc.at[pl.ds(s * STRIPE + nfull * SCHUNK, rem)])
    plsc.subcore_barrier()

    def fone(g, carry):
        ones_v[pl.ds(g * 16, 16)] = jnp.ones((16,), jnp.float32)
        return carry

    lax.fori_loop(0, VGRP, fone, 0)

    base = s * BONDS_PER_T

    def chunk(j, carry):
        off = base + j * SCHUNK
        pltpu.sync_copy(idx_hbm.at[pl.ds(off, SCHUNK)], idx_v)
        pltpu.sync_copy(bnew_hbm.at[pl.ds(off, SCHUNK)], rows_v)

        def remap(g, carry2):
            v = idx_v[pl.ds(g * 16, 16)]
            local = v - lo
            inr = (local >= 0) & (local < ATOMS_PER_SC)
            trash = ATOMS_PER_SC + (v & 127)
            lidx_v[pl.ds(g * 16, 16)] = jnp.where(inr, local, trash)
            return carry2

        lax.fori_loop(0, VGRP, remap, 0)
        pltpu.sync_copy(rows_v, feat_acc.at[lidx_v], add=True)
        pltpu.sync_copy(ones_v, cnt_acc.at[lidx_v], add=True)
        return carry

    lax.fori_loop(0, SCHUNKS, chunk, 0)
    plsc.subcore_barrier()

    out_off = lo + s * STRIPE

    @pl.when(s < NS - 1)
    def _():
        pltpu.sync_copy(feat_acc.at[pl.ds(s * STRIPE, STRIPE)],
                        seg_hbm.at[pl.ds(out_off, STRIPE)])
        pltpu.sync_copy(cnt_acc.at[pl.ds(s * STRIPE, STRIPE)],
                        cnt_hbm.at[pl.ds(out_off, STRIPE)])

    @pl.when(s == NS - 1)
    def _():
        pltpu.sync_copy(feat_acc.at[pl.ds(s * STRIPE, LAST_STRIPE)],
                        seg_hbm.at[pl.ds(out_off, LAST_STRIPE)])
        pltpu.sync_copy(cnt_acc.at[pl.ds(s * STRIPE, LAST_STRIPE)],
                        cnt_hbm.at[pl.ds(out_off, LAST_STRIPE)])


def _sc_scatter(bonds_new, idx1):
    mesh = plsc.VectorSubcoreMesh(core_axis_name="c", subcore_axis_name="s")
    f = pl.kernel(
        _scatter_body,
        out_type=[jax.ShapeDtypeStruct((NA, D), jnp.float32),
                  jax.ShapeDtypeStruct((NA,), jnp.float32)],
        mesh=mesh,
        compiler_params=pltpu.CompilerParams(use_tc_tiling_on_sc=False),
        scratch_types=[pltpu.VMEM((SCHUNK,), jnp.int32),
                       pltpu.VMEM((SCHUNK,), jnp.int32),
                       pltpu.VMEM((SCHUNK, D), jnp.float32),
                       pltpu.VMEM((SCHUNK,), jnp.float32),
                       pltpu.VMEM_SHARED((ACC_ROWS, D), jnp.float32),
                       pltpu.VMEM_SHARED((ACC_ROWS,), jnp.float32),
                       pltpu.SemaphoreType.DMA],
    )
    return f(bonds_new, idx1)


def _softplus(x):
    return jnp.maximum(x, 0.0) + jnp.log1p(jnp.exp(-jnp.abs(x)))


EBLK = 6400
EGRID = NB // EBLK


def _edge_body(a1, a2, bd, st, W1, b1, W2, b2, W3, b3, out, acc):
    i = pl.program_id(0)
    x = jnp.concatenate([a1[...], a2[...], bd[...]], axis=1)
    c0 = jnp.dot(st[...], W1[96:128, :],
                 preferred_element_type=jnp.float32) + b1[...]
    h = jnp.dot(x, W1[0:96, :], preferred_element_type=jnp.float32) + c0
    h = _softplus(h)
    h = _softplus(jnp.dot(h, W2[...], preferred_element_type=jnp.float32)
                  + b2[...])
    h = _softplus(jnp.dot(h, W3[...], preferred_element_type=jnp.float32)
                  + b3[...])
    out[...] = h

    @pl.when(i == 0)
    def _():
        acc[...] = jnp.zeros_like(acc)

    acc[...] += jnp.sum(h.reshape(8, EBLK // 8, D), axis=1)


def _edge_mlp(a1, a2, bonds, state, W1, b1, W2, b2, W3, b3):
    full = lambda shape: pl.BlockSpec(shape, lambda i: (0, 0))
    return pl.pallas_call(
        _edge_body,
        grid=(EGRID,),
        in_specs=[
            pl.BlockSpec((EBLK, D), lambda i: (i, 0)),
            pl.BlockSpec((EBLK, D), lambda i: (i, 0)),
            pl.BlockSpec((EBLK, D), lambda i: (i, 0)),
            full((1, D)),
            full((128, 64)), full((1, 64)),
            full((64, 64)), full((1, 64)),
            full((64, 32)), full((1, 32)),
        ],
        out_specs=[
            pl.BlockSpec((EBLK, D), lambda i: (i, 0)),
            pl.BlockSpec((8, D), lambda i: (0, 0)),
        ],
        out_shape=[jax.ShapeDtypeStruct((NB, D), jnp.float32),
                   jax.ShapeDtypeStruct((8, D), jnp.float32)],
    )(a1, a2, bonds, state, W1, b1, W2, b2, W3, b3)


ABLK = 1000
AGRID = NA // ABLK


def _atom_body(seg, cnt, at, st, W1, b1, W2, b2, W3, b3, out, acc):
    i = pl.program_id(0)
    b2a = seg[...] / cnt[...]
    x = jnp.concatenate([b2a, at[...]], axis=1)
    c0 = jnp.dot(st[...], W1[64:96, :],
                 preferred_element_type=jnp.float32) + b1[...]
    h = jnp.dot(x, W1[0:64, :], preferred_element_type=jnp.float32) + c0
    h = _softplus(h)
    h = _softplus(jnp.dot(h, W2[...], preferred_element_type=jnp.float32)
                  + b2[...])
    h = _softplus(jnp.dot(h, W3[...], preferred_element_type=jnp.float32)
                  + b3[...])
    out[...] = h

    @pl.when(i == 0)
    def _():
        acc[...] = jnp.zeros_like(acc)

    acc[...] += jnp.sum(h.reshape(8, ABLK // 8, D), axis=1)


def _atom_mlp(seg, cnt, atoms, state, W1, b1, W2, b2, W3, b3):
    full = lambda shape: pl.BlockSpec(shape, lambda i: (0, 0))
    return pl.pallas_call(
        _atom_body,
        grid=(AGRID,),
        in_specs=[
            pl.BlockSpec((ABLK, D), lambda i: (i, 0)),
            pl.BlockSpec((ABLK, 1), lambda i: (i, 0)),
            pl.BlockSpec((ABLK, D), lambda i: (i, 0)),
            full((1, D)),
            full((96, 64)), full((1, 64)),
            full((64, 64)), full((1, 64)),
            full((64, 32)), full((1, 32)),
        ],
        out_specs=[
            pl.BlockSpec((ABLK, D), lambda i: (i, 0)),
            pl.BlockSpec((8, D), lambda i: (0, 0)),
        ],
        out_shape=[jax.ShapeDtypeStruct((NA, D), jnp.float32),
                   jax.ShapeDtypeStruct((8, D), jnp.float32)],
    )(seg, cnt, atoms, state, W1, b1, W2, b2, W3, b3)


def _state_body(bacc, aacc, st, W1, b1, W2, b2, W3, b3, out):
    b2s = jnp.sum(bacc[...], axis=0, keepdims=True) / NB
    a2s = jnp.sum(aacc[...], axis=0, keepdims=True) / NA
    c0 = jnp.dot(st[...], W1[64:96, :],
                 preferred_element_type=jnp.float32) + b1[...]
    h = (jnp.dot(b2s, W1[0:32, :], preferred_element_type=jnp.float32)
         + jnp.dot(a2s, W1[32:64, :], preferred_element_type=jnp.float32)
         + c0)
    h = _softplus(h)
    h = _softplus(jnp.dot(h, W2[...], preferred_element_type=jnp.float32)
                  + b2[...])
    h = _softplus(jnp.dot(h, W3[...], preferred_element_type=jnp.float32)
                  + b3[...])
    out[...] = h


def _state_mlp(bacc, aacc, state, W1, b1, W2, b2, W3, b3):
    return pl.pallas_call(
        _state_body,
        out_shape=jax.ShapeDtypeStruct((1, D), jnp.float32),
    )(bacc, aacc, state, W1, b1, W2, b2, W3, b3)


def kernel(bonds, bond_atom_1, bond_atom_2, atoms, state,
           e_W1, e_b1, e_W2, e_b2, e_W3, e_b3,
           v_W1, v_b1, v_W2, v_b2, v_W3, v_b3,
           u_W1, u_b1, u_W2, u_b2, u_W3, u_b3):
    a1, a2 = _sc_gather(atoms, bond_atom_1, bond_atom_2)
    bonds_new, bacc = _edge_mlp(
        a1, a2, bonds, state,
        e_W1, e_b1.reshape(1, 64), e_W2, e_b2.reshape(1, 64),
        e_W3, e_b3.reshape(1, 32))
    seg, cnt = _sc_scatter(bonds_new, bond_atom_1)
    atoms_new, aacc = _atom_mlp(
        seg, cnt.reshape(NA, 1), atoms, state,
        v_W1, v_b1.reshape(1, 64), v_W2, v_b2.reshape(1, 64),
        v_W3, v_b3.reshape(1, 32))
    state_new = _state_mlp(
        bacc, aacc, state,
        u_W1, u_b1.reshape(1, 64), u_W2, u_b2.reshape(1, 64),
        u_W3, u_b3.reshape(1, 32))
    return (bonds_new, atoms_new, state_new)


# speedup vs baseline: 3.4859x; 1.0755x over previous
import functools

import jax
import jax.numpy as jnp
from jax import lax
from jax.experimental import pallas as pl
from jax.experimental.pallas import tpu as pltpu
from jax.experimental.pallas import tpu_sc as plsc

NB = 1_600_000
NA = 100_000
D = 32
NC = 2
NS = 16
NW = NC * NS

GCHUNK = 2000
BONDS_PER_W = NB // NW
GCHUNKS = BONDS_PER_W // GCHUNK


def _gather_body(atoms_hbm, idx1_hbm, idx2_hbm, a1_hbm, a2_hbm,
                 idx_v, rows_v, sem):
    c = lax.axis_index("c")
    s = lax.axis_index("s")
    wid = s * NC + c
    base = wid * BONDS_PER_W

    def chunk(j, carry):
        off = base + j * GCHUNK
        pltpu.sync_copy(idx1_hbm.at[pl.ds(off, GCHUNK)], idx_v)
        pltpu.async_copy(atoms_hbm.at[idx_v], rows_v, sem).wait()
        pltpu.sync_copy(rows_v, a1_hbm.at[pl.ds(off, GCHUNK)])
        pltpu.sync_copy(idx2_hbm.at[pl.ds(off, GCHUNK)], idx_v)
        pltpu.async_copy(atoms_hbm.at[idx_v], rows_v, sem).wait()
        pltpu.sync_copy(rows_v, a2_hbm.at[pl.ds(off, GCHUNK)])
        return carry

    lax.fori_loop(0, GCHUNKS, chunk, 0)


def _sc_gather(atoms, idx1, idx2):
    mesh = plsc.VectorSubcoreMesh(core_axis_name="c", subcore_axis_name="s")
    f = pl.kernel(
        _gather_body,
        out_type=[jax.ShapeDtypeStruct((NB, D), jnp.float32),
                  jax.ShapeDtypeStruct((NB, D), jnp.float32)],
        mesh=mesh,
        compiler_params=pltpu.CompilerParams(use_tc_tiling_on_sc=False),
        scratch_types=[pltpu.VMEM((GCHUNK,), jnp.int32),
                       pltpu.VMEM((GCHUNK, D), jnp.float32),
                       pltpu.SemaphoreType.DMA],
    )
    return f(atoms, idx1, idx2)


ATOMS_PER_SC = NA // NC
ACC_ROWS = 50176
STRIPE = ACC_ROWS // NS
LAST_STRIPE = ATOMS_PER_SC - (NS - 1) * STRIPE
SCHUNK = 400
BONDS_PER_T = NB // NS
SCHUNKS = BONDS_PER_T // SCHUNK
VGRP = SCHUNK // 16


def _scatter_body(bnew_hbm, idx_hbm, seg_hbm, cnt_hbm,
                  idx_v, lidx_v, rows_v, ones_v, feat_acc, cnt_acc, sem):
    c = lax.axis_index("c")
    s = lax.axis_index("s")
    lo = c * ATOMS_PER_SC

    def zrow(i, carry):
        rows_v[i, pl.ds(0, 16)] = jnp.zeros((16,), jnp.float32)
        rows_v[i, pl.ds(16, 16)] = jnp.zeros((16,), jnp.float32)
        return carry

    lax.fori_loop(0, SCHUNK, zrow, 0)

    def zone(g, carry):
        ones_v[pl.ds(g * 16, 16)] = jnp.zeros((16,), jnp.float32)
        return carry

    lax.fori_loop(0, VGRP, zone, 0)

    nfull = STRIPE // SCHUNK
    rem = STRIPE - nfull * SCHUNK

    def zcopy(k, carry):
        pltpu.sync_copy(rows_v,
                        feat_acc.at[pl.ds(s * STRIPE + k * SCHUNK, SCHUNK)])
        pltpu.sync_copy(ones_v,
                        cnt_acc.at[pl.ds(s * STRIPE + k * SCHUNK, SCHUNK)])
        return carry

    lax.fori_loop(0, nfull, zcopy, 0)
    pltpu.sync_copy(rows_v.at[pl.ds(0, rem)],
                    feat_acc.at[pl.ds(s * STRIPE + nfull * SCHUNK, rem)])
    pltpu.sync_copy(ones_v.at[pl.ds(0, rem)],
                    cnt_acc.at[pl.ds(s * STRIPE + nfull * SCHUNK, rem)])
    plsc.subcore_barrier()

    def fone(g, carry):
        ones_v[pl.ds(g * 16, 16)] = jnp.ones((16,), jnp.float32)
        return carry

    lax.fori_loop(0, VGRP, fone, 0)

    base = s * BONDS_PER_T

    def chunk(j, carry):
        off = base + j * SCHUNK
        pltpu.sync_copy(idx_hbm.at[pl.ds(off, SCHUNK)], idx_v)
        pltpu.sync_copy(bnew_hbm.at[pl.ds(off, SCHUNK)], rows_v)

        def remap(g, carry2):
            v = idx_v[pl.ds(g * 16, 16)]
            local = v - lo
            inr = (local >= 0) & (local < ATOMS_PER_SC)
            trash = ATOMS_PER_SC + (v & 127)
            lidx_v[pl.ds(g * 16, 16)] = jnp.where(inr, local, trash)
            return carry2

        lax.fori_loop(0, VGRP, remap, 0)
        pltpu.sync_copy(rows_v, feat_acc.at[lidx_v], add=True)
        pltpu.sync_copy(ones_v, cnt_acc.at[lidx_v], add=True)
        return carry

    lax.fori_loop(0, SCHUNKS, chunk, 0)
    plsc.subcore_barrier()

    out_off = lo + s * STRIPE

    @pl.when(s < NS - 1)
    def _():
        pltpu.sync_copy(feat_acc.at[pl.ds(s * STRIPE, STRIPE)],
                        seg_hbm.at[pl.ds(out_off, STRIPE)])
        pltpu.sync_copy(cnt_acc.at[pl.ds(s * STRIPE, STRIPE)],
                        cnt_hbm.at[pl.ds(out_off, STRIPE)])

    @pl.when(s == NS - 1)
    def _():
        pltpu.sync_copy(feat_acc.at[pl.ds(s * STRIPE, LAST_STRIPE)],
                        seg_hbm.at[pl.ds(out_off, LAST_STRIPE)])
        pltpu.sync_copy(cnt_acc.at[pl.ds(s * STRIPE, LAST_STRIPE)],
                        cnt_hbm.at[pl.ds(out_off, LAST_STRIPE)])


def _sc_scatter(bonds_new, idx1):
    mesh = plsc.VectorSubcoreMesh(core_axis_name="c", subcore_axis_name="s")
    f = pl.kernel(
        _scatter_body,
        out_type=[jax.ShapeDtypeStruct((NA, D), jnp.float32),
                  jax.ShapeDtypeStruct((NA,), jnp.float32)],
        mesh=mesh,
        compiler_params=pltpu.CompilerParams(use_tc_tiling_on_sc=False),
        scratch_types=[pltpu.VMEM((SCHUNK,), jnp.int32),
                       pltpu.VMEM((SCHUNK,), jnp.int32),
                       pltpu.VMEM((SCHUNK, D), jnp.float32),
                       pltpu.VMEM((SCHUNK,), jnp.float32),
                       pltpu.VMEM_SHARED((ACC_ROWS, D), jnp.float32),
                       pltpu.VMEM_SHARED((ACC_ROWS,), jnp.float32),
                       pltpu.SemaphoreType.DMA],
    )
    return f(bonds_new, idx1)


def _softplus(x):
    return jnp.maximum(x, 0.0) + jnp.log(1.0 + jnp.exp(-jnp.abs(x)))


EBLK = 6400
EGRID = NB // EBLK


def _edge_body(a1, a2, bd, st, W1, b1, W2, b2, W3, b3, out, acc):
    i = pl.program_id(0)
    x = jnp.concatenate([a1[...], a2[...], bd[...]], axis=1)
    c0 = jnp.dot(st[...], W1[96:128, :],
                 preferred_element_type=jnp.float32) + b1[...]
    h = jnp.dot(x, W1[0:96, :], preferred_element_type=jnp.float32) + c0
    h = _softplus(h)
    h = _softplus(jnp.dot(h, W2[...], preferred_element_type=jnp.float32)
                  + b2[...])
    h = _softplus(jnp.dot(h, W3[...], preferred_element_type=jnp.float32)
                  + b3[...])
    out[...] = h

    @pl.when(i == 0)
    def _():
        acc[...] = jnp.zeros_like(acc)

    acc[...] += jnp.sum(h.reshape(8, EBLK // 8, D), axis=1)


def _edge_mlp(a1, a2, bonds, state, W1, b1, W2, b2, W3, b3):
    full = lambda shape: pl.BlockSpec(shape, lambda i: (0, 0))
    return pl.pallas_call(
        _edge_body,
        grid=(EGRID,),
        in_specs=[
            pl.BlockSpec((EBLK, D), lambda i: (i, 0)),
            pl.BlockSpec((EBLK, D), lambda i: (i, 0)),
            pl.BlockSpec((EBLK, D), lambda i: (i, 0)),
            full((1, D)),
            full((128, 64)), full((1, 64)),
            full((64, 64)), full((1, 64)),
            full((64, 32)), full((1, 32)),
        ],
        out_specs=[
            pl.BlockSpec((EBLK, D), lambda i: (i, 0)),
            pl.BlockSpec((8, D), lambda i: (0, 0)),
        ],
        out_shape=[jax.ShapeDtypeStruct((NB, D), jnp.float32),
                   jax.ShapeDtypeStruct((8, D), jnp.float32)],
    )(a1, a2, bonds, state, W1, b1, W2, b2, W3, b3)


ABLK = 1000
AGRID = NA // ABLK


def _atom_body(seg, cnt, at, st, W1, b1, W2, b2, W3, b3, out, acc):
    i = pl.program_id(0)
    b2a = seg[...] / cnt[...]
    x = jnp.concatenate([b2a, at[...]], axis=1)
    c0 = jnp.dot(st[...], W1[64:96, :],
                 preferred_element_type=jnp.float32) + b1[...]
    h = jnp.dot(x, W1[0:64, :], preferred_element_type=jnp.float32) + c0
    h = _softplus(h)
    h = _softplus(jnp.dot(h, W2[...], preferred_element_type=jnp.float32)
                  + b2[...])
    h = _softplus(jnp.dot(h, W3[...], preferred_element_type=jnp.float32)
                  + b3[...])
    out[...] = h

    @pl.when(i == 0)
    def _():
        acc[...] = jnp.zeros_like(acc)

    acc[...] += jnp.sum(h.reshape(8, ABLK // 8, D), axis=1)


def _atom_mlp(seg, cnt, atoms, state, W1, b1, W2, b2, W3, b3):
    full = lambda shape: pl.BlockSpec(shape, lambda i: (0, 0))
    return pl.pallas_call(
        _atom_body,
        grid=(AGRID,),
        in_specs=[
            pl.BlockSpec((ABLK, D), lambda i: (i, 0)),
            pl.BlockSpec((ABLK, 1), lambda i: (i, 0)),
            pl.BlockSpec((ABLK, D), lambda i: (i, 0)),
            full((1, D)),
            full((96, 64)), full((1, 64)),
            full((64, 64)), full((1, 64)),
            full((64, 32)), full((1, 32)),
        ],
        out_specs=[
            pl.BlockSpec((ABLK, D), lambda i: (i, 0)),
            pl.BlockSpec((8, D), lambda i: (0, 0)),
        ],
        out_shape=[jax.ShapeDtypeStruct((NA, D), jnp.float32),
                   jax.ShapeDtypeStruct((8, D), jnp.float32)],
    )(seg, cnt, atoms, state, W1, b1, W2, b2, W3, b3)


def _state_body(bacc, aacc, st, W1, b1, W2, b2, W3, b3, out):
    b2s = jnp.sum(bacc[...], axis=0, keepdims=True) / NB
    a2s = jnp.sum(aacc[...], axis=0, keepdims=True) / NA
    c0 = jnp.dot(st[...], W1[64:96, :],
                 preferred_element_type=jnp.float32) + b1[...]
    h = (jnp.dot(b2s, W1[0:32, :], preferred_element_type=jnp.float32)
         + jnp.dot(a2s, W1[32:64, :], preferred_element_type=jnp.float32)
         + c0)
    h = _softplus(h)
    h = _softplus(jnp.dot(h, W2[...], preferred_element_type=jnp.float32)
                  + b2[...])
    h = _softplus(jnp.dot(h, W3[...], preferred_element_type=jnp.float32)
                  + b3[...])
    out[...] = h


def _state_mlp(bacc, aacc, state, W1, b1, W2, b2, W3, b3):
    return pl.pallas_call(
        _state_body,
        out_shape=jax.ShapeDtypeStruct((1, D), jnp.float32),
    )(bacc, aacc, state, W1, b1, W2, b2, W3, b3)


def kernel(bonds, bond_atom_1, bond_atom_2, atoms, state,
           e_W1, e_b1, e_W2, e_b2, e_W3, e_b3,
           v_W1, v_b1, v_W2, v_b2, v_W3, v_b3,
           u_W1, u_b1, u_W2, u_b2, u_W3, u_b3):
    a1, a2 = _sc_gather(atoms, bond_atom_1, bond_atom_2)
    bonds_new, bacc = _edge_mlp(
        a1, a2, bonds, state,
        e_W1, e_b1.reshape(1, 64), e_W2, e_b2.reshape(1, 64),
        e_W3, e_b3.reshape(1, 32))
    seg, cnt = _sc_scatter(bonds_new, bond_atom_1)
    atoms_new, aacc = _atom_mlp(
        seg, cnt.reshape(NA, 1), atoms, state,
        v_W1, v_b1.reshape(1, 64), v_W2, v_b2.reshape(1, 64),
        v_W3, v_b3.reshape(1, 32))
    state_new = _state_mlp(
        bacc, aacc, state,
        u_W1, u_b1.reshape(1, 64), u_W2, u_b2.reshape(1, 64),
        u_W3, u_b3.reshape(1, 32))
    return (bonds_new, atoms_new, state_new)


# speedup vs baseline: 6.0143x; 1.7253x over previous
import functools

import jax
import jax.numpy as jnp
from jax import lax
from jax.experimental import pallas as pl
from jax.experimental.pallas import tpu as pltpu
from jax.experimental.pallas import tpu_sc as plsc

NB = 1_600_000
NA = 100_000
D = 32
NC = 2
NS = 16
NW = NC * NS

GCHUNK = 2000
BONDS_PER_W = NB // NW
GCHUNKS = BONDS_PER_W // GCHUNK


def _gather_body(atoms_hbm, idx1_hbm, idx2_hbm, a1_hbm, a2_hbm,
                 idx_v, rows_v, sem):
    c = lax.axis_index("c")
    s = lax.axis_index("s")
    wid = s * NC + c
    base = wid * BONDS_PER_W

    def chunk(j, carry):
        off = base + j * GCHUNK
        pltpu.sync_copy(idx1_hbm.at[pl.ds(off, GCHUNK)], idx_v)
        pltpu.async_copy(atoms_hbm.at[idx_v], rows_v, sem).wait()
        pltpu.sync_copy(rows_v, a1_hbm.at[pl.ds(off, GCHUNK)])
        pltpu.sync_copy(idx2_hbm.at[pl.ds(off, GCHUNK)], idx_v)
        pltpu.async_copy(atoms_hbm.at[idx_v], rows_v, sem).wait()
        pltpu.sync_copy(rows_v, a2_hbm.at[pl.ds(off, GCHUNK)])
        return carry

    lax.fori_loop(0, GCHUNKS, chunk, 0)


def _sc_gather(atoms, idx1, idx2):
    mesh = plsc.VectorSubcoreMesh(core_axis_name="c", subcore_axis_name="s")
    f = pl.kernel(
        _gather_body,
        out_type=[jax.ShapeDtypeStruct((NB, D), jnp.float32),
                  jax.ShapeDtypeStruct((NB, D), jnp.float32)],
        mesh=mesh,
        compiler_params=pltpu.CompilerParams(use_tc_tiling_on_sc=False),
        scratch_types=[pltpu.VMEM((GCHUNK,), jnp.int32),
                       pltpu.VMEM((GCHUNK, D), jnp.float32),
                       pltpu.SemaphoreType.DMA],
    )
    return f(atoms, idx1, idx2)


ATOMS_PER_SC = NA // NC
ACC_ROWS = 50176
STRIPE = ACC_ROWS // NS
LAST_STRIPE = ATOMS_PER_SC - (NS - 1) * STRIPE
SCHUNK = 400
BONDS_PER_T = NB // NS
SCHUNKS = BONDS_PER_T // SCHUNK
VGRP = SCHUNK // 16


def _scatter_body(bnew_hbm, idx_hbm, seg_hbm, cnt_hbm,
                  idx_v, lidx_v, rows_v, ones_v, feat_acc, cnt_acc, sem):
    c = lax.axis_index("c")
    s = lax.axis_index("s")
    lo = c * ATOMS_PER_SC

    def zrow(i, carry):
        rows_v[i, pl.ds(0, 16)] = jnp.zeros((16,), jnp.float32)
        rows_v[i, pl.ds(16, 16)] = jnp.zeros((16,), jnp.float32)
        return carry

    lax.fori_loop(0, SCHUNK, zrow, 0)

    def zone(g, carry):
        ones_v[pl.ds(g * 16, 16)] = jnp.zeros((16,), jnp.float32)
        return carry

    lax.fori_loop(0, VGRP, zone, 0)

    nfull = STRIPE // SCHUNK
    rem = STRIPE - nfull * SCHUNK

    def zcopy(k, carry):
        pltpu.sync_copy(rows_v,
                        feat_acc.at[pl.ds(s * STRIPE + k * SCHUNK, SCHUNK)])
        pltpu.sync_copy(ones_v,
                        cnt_acc.at[pl.ds(s * STRIPE + k * SCHUNK, SCHUNK)])
        return carry

    lax.fori_loop(0, nfull, zcopy, 0)
    pltpu.sync_copy(rows_v.at[pl.ds(0, rem)],
                    feat_acc.at[pl.ds(s * STRIPE + nfull * SCHUNK, rem)])
    pltpu.sync_copy(ones_v.at[pl.ds(0, rem)],
                    cnt_acc.at[pl.ds(s * STRIPE + nfull * SCHUNK, rem)])
    plsc.subcore_barrier()

    def fone(g, carry):
        ones_v[pl.ds(g * 16, 16)] = jnp.ones((16,), jnp.float32)
        return carry

    lax.fori_loop(0, VGRP, fone, 0)

    base = s * BONDS_PER_T

    def chunk(j, carry):
        off = base + j * SCHUNK
        pltpu.sync_copy(idx_hbm.at[pl.ds(off, SCHUNK)], idx_v)
        pltpu.sync_copy(bnew_hbm.at[pl.ds(off, SCHUNK)], rows_v)

        def remap(g, carry2):
            v = idx_v[pl.ds(g * 16, 16)]
            local = v - lo
            inr = (local >= 0) & (local < ATOMS_PER_SC)
            trash = ATOMS_PER_SC + (v & 127)
            lidx_v[pl.ds(g * 16, 16)] = jnp.where(inr, local, trash)
            return carry2

        lax.fori_loop(0, VGRP, remap, 0)
        pltpu.sync_copy(rows_v, feat_acc.at[lidx_v], add=True)
        pltpu.sync_copy(ones_v, cnt_acc.at[lidx_v], add=True)
        return carry

    lax.fori_loop(0, SCHUNKS, chunk, 0)
    plsc.subcore_barrier()

    out_off = lo + s * STRIPE

    @pl.when(s < NS - 1)
    def _():
        pltpu.sync_copy(feat_acc.at[pl.ds(s * STRIPE, STRIPE)],
                        seg_hbm.at[pl.ds(out_off, STRIPE)])
        pltpu.sync_copy(cnt_acc.at[pl.ds(s * STRIPE, STRIPE)],
                        cnt_hbm.at[pl.ds(out_off, STRIPE)])

    @pl.when(s == NS - 1)
    def _():
        pltpu.sync_copy(feat_acc.at[pl.ds(s * STRIPE, LAST_STRIPE)],
                        seg_hbm.at[pl.ds(out_off, LAST_STRIPE)])
        pltpu.sync_copy(cnt_acc.at[pl.ds(s * STRIPE, LAST_STRIPE)],
                        cnt_hbm.at[pl.ds(out_off, LAST_STRIPE)])


def _sc_scatter(bonds_new, idx1):
    mesh = plsc.VectorSubcoreMesh(core_axis_name="c", subcore_axis_name="s")
    f = pl.kernel(
        _scatter_body,
        out_type=[jax.ShapeDtypeStruct((NA, D), jnp.float32),
                  jax.ShapeDtypeStruct((NA,), jnp.float32)],
        mesh=mesh,
        compiler_params=pltpu.CompilerParams(use_tc_tiling_on_sc=False),
        scratch_types=[pltpu.VMEM((SCHUNK,), jnp.int32),
                       pltpu.VMEM((SCHUNK,), jnp.int32),
                       pltpu.VMEM((SCHUNK, D), jnp.float32),
                       pltpu.VMEM((SCHUNK,), jnp.float32),
                       pltpu.VMEM_SHARED((ACC_ROWS, D), jnp.float32),
                       pltpu.VMEM_SHARED((ACC_ROWS,), jnp.float32),
                       pltpu.SemaphoreType.DMA],
    )
    return f(bonds_new, idx1)


def _softplus(x):
    return jnp.maximum(x, 0.0) + jnp.log(1.0 + jnp.exp(-jnp.abs(x)))


EROWS = NB // 4
EBLK = 1600
EGRID = EROWS // EBLK


def _edge_body(a1p, a2p, bdp, st, W1a, W1b, W1c, W1d, b1, b2p, b3p,
               W2bd, W3bd, out, acc):
    i = pl.program_id(0)
    c0 = jnp.dot(st[...], W1d[...], preferred_element_type=jnp.float32) \
        + b1[...]
    c0p = jnp.concatenate([c0, c0, c0, c0], axis=1)
    h = (jnp.dot(a1p[...], W1a[...], preferred_element_type=jnp.float32)
         + jnp.dot(a2p[...], W1b[...], preferred_element_type=jnp.float32)
         + jnp.dot(bdp[...], W1c[...], preferred_element_type=jnp.float32)
         + c0p)
    h = _softplus(h)
    h = _softplus(jnp.dot(h, W2bd[...], preferred_element_type=jnp.float32)
                  + b2p[...])
    h = _softplus(jnp.dot(h, W3bd[...], preferred_element_type=jnp.float32)
                  + b3p[...])
    out[...] = h

    @pl.when(i == 0)
    def _():
        acc[...] = jnp.zeros_like(acc)

    acc[...] += jnp.sum(h.reshape(8, EBLK // 8, 128), axis=1)


def _edge_mlp(a1p, a2p, bdp, state, W1a, W1b, W1c, W1d, b1, b2p, b3p,
              W2bd, W3bd):
    full = lambda shape: pl.BlockSpec(shape, lambda i: (0, 0))
    return pl.pallas_call(
        _edge_body,
        grid=(EGRID,),
        in_specs=[
            pl.BlockSpec((EBLK, 128), lambda i: (i, 0)),
            pl.BlockSpec((EBLK, 128), lambda i: (i, 0)),
            pl.BlockSpec((EBLK, 128), lambda i: (i, 0)),
            full((1, D)),
            full((128, 256)), full((128, 256)), full((128, 256)),
            full((32, 64)), full((1, 64)), full((1, 256)), full((1, 128)),
            full((256, 256)), full((256, 128)),
        ],
        out_specs=[
            pl.BlockSpec((EBLK, 128), lambda i: (i, 0)),
            pl.BlockSpec((8, 128), lambda i: (0, 0)),
        ],
        out_shape=[jax.ShapeDtypeStruct((EROWS, 128), jnp.float32),
                   jax.ShapeDtypeStruct((8, 128), jnp.float32)],
    )(a1p, a2p, bdp, state, W1a, W1b, W1c, W1d, b1, b2p, b3p, W2bd, W3bd)


ABLK = 1000
AGRID = NA // ABLK


def _atom_body(seg, cnt, at, st, W1, b1, W2, b2, W3, b3, out, acc):
    i = pl.program_id(0)
    b2a = seg[...] / cnt[...]
    x = jnp.concatenate([b2a, at[...]], axis=1)
    c0 = jnp.dot(st[...], W1[64:96, :],
                 preferred_element_type=jnp.float32) + b1[...]
    h = jnp.dot(x, W1[0:64, :], preferred_element_type=jnp.float32) + c0
    h = _softplus(h)
    h = _softplus(jnp.dot(h, W2[...], preferred_element_type=jnp.float32)
                  + b2[...])
    h = _softplus(jnp.dot(h, W3[...], preferred_element_type=jnp.float32)
                  + b3[...])
    out[...] = h

    @pl.when(i == 0)
    def _():
        acc[...] = jnp.zeros_like(acc)

    acc[...] += jnp.sum(h.reshape(8, ABLK // 8, D), axis=1)


def _atom_mlp(seg, cnt, atoms, state, W1, b1, W2, b2, W3, b3):
    full = lambda shape: pl.BlockSpec(shape, lambda i: (0, 0))
    return pl.pallas_call(
        _atom_body,
        grid=(AGRID,),
        in_specs=[
            pl.BlockSpec((ABLK, D), lambda i: (i, 0)),
            pl.BlockSpec((ABLK, 1), lambda i: (i, 0)),
            pl.BlockSpec((ABLK, D), lambda i: (i, 0)),
            full((1, D)),
            full((96, 64)), full((1, 64)),
            full((64, 64)), full((1, 64)),
            full((64, 32)), full((1, 32)),
        ],
        out_specs=[
            pl.BlockSpec((ABLK, D), lambda i: (i, 0)),
            pl.BlockSpec((8, D), lambda i: (0, 0)),
        ],
        out_shape=[jax.ShapeDtypeStruct((NA, D), jnp.float32),
                   jax.ShapeDtypeStruct((8, D), jnp.float32)],
    )(seg, cnt, atoms, state, W1, b1, W2, b2, W3, b3)


def _state_body(bacc, aacc, st, W1, b1, W2, b2, W3, b3, out):
    bp = bacc[...]
    bsum = (bp[:, 0:32] + bp[:, 32:64] + bp[:, 64:96] + bp[:, 96:128])
    b2s = jnp.sum(bsum, axis=0, keepdims=True) / NB
    a2s = jnp.sum(aacc[...], axis=0, keepdims=True) / NA
    c0 = jnp.dot(st[...], W1[64:96, :],
                 preferred_element_type=jnp.float32) + b1[...]
    h = (jnp.dot(b2s, W1[0:32, :], preferred_element_type=jnp.float32)
         + jnp.dot(a2s, W1[32:64, :], preferred_element_type=jnp.float32)
         + c0)
    h = _softplus(h)
    h = _softplus(jnp.dot(h, W2[...], preferred_element_type=jnp.float32)
                  + b2[...])
    h = _softplus(jnp.dot(h, W3[...], preferred_element_type=jnp.float32)
                  + b3[...])
    out[...] = h


def _state_mlp(bacc, aacc, state, W1, b1, W2, b2, W3, b3):
    return pl.pallas_call(
        _state_body,
        out_shape=jax.ShapeDtypeStruct((1, D), jnp.float32),
    )(bacc, aacc, state, W1, b1, W2, b2, W3, b3)


def kernel(bonds, bond_atom_1, bond_atom_2, atoms, state,
           e_W1, e_b1, e_W2, e_b2, e_W3, e_b3,
           v_W1, v_b1, v_W2, v_b2, v_W3, v_b3,
           u_W1, u_b1, u_W2, u_b2, u_W3, u_b3):
    a1, a2 = _sc_gather(atoms, bond_atom_1, bond_atom_2)
    eye4 = jnp.eye(4, dtype=jnp.float32)
    bnp, bacc = _edge_mlp(
        a1.reshape(EROWS, 128), a2.reshape(EROWS, 128),
        bonds.reshape(EROWS, 128), state,
        jnp.kron(eye4, e_W1[0:32, :]), jnp.kron(eye4, e_W1[32:64, :]),
        jnp.kron(eye4, e_W1[64:96, :]), e_W1[96:128, :],
        e_b1.reshape(1, 64),
        jnp.tile(e_b2, 4).reshape(1, 256), jnp.tile(e_b3, 4).reshape(1, 128),
        jnp.kron(eye4, e_W2), jnp.kron(eye4, e_W3))
    bonds_new = bnp.reshape(NB, D)
    seg, cnt = _sc_scatter(bonds_new, bond_atom_1)
    atoms_new, aacc = _atom_mlp(
        seg, cnt.reshape(NA, 1), atoms, state,
        v_W1, v_b1.reshape(1, 64), v_W2, v_b2.reshape(1, 64),
        v_W3, v_b3.reshape(1, 32))
    state_new = _state_mlp(
        bacc, aacc, state,
        u_W1, u_b1.reshape(1, 64), u_W2, u_b2.reshape(1, 64),
        u_W3, u_b3.reshape(1, 32))
    return (bonds_new, atoms_new, state_new)
